# baseline (speedup 1.0000x reference)
import functools

import jax
import jax.numpy as jnp
from jax import lax
from jax.experimental import pallas as pl
from jax.experimental.pallas import tpu as pltpu
from jax.experimental.pallas import tpu_sc as plsc

B, C, N, K = 1, 128, 50000, 9
C_OUT = 256

NC, NS = 2, 16
NW = NC * NS
NPT = 1568
TAIL = N - 31 * NPT
G = 8
CHUNKS = NPT // G
TAIL_CH = TAIL // G
NBUF = 4
NB = 2048
NBLK = -(-N // NB)
EPS = 1e-5


def _sc_mrconv(xT, e2):
    mesh = plsc.VectorSubcoreMesh(core_axis_name="c", subcore_axis_name="s")

    @functools.partial(
        pl.kernel,
        out_type=jax.ShapeDtypeStruct((N, C), jnp.float32),
        mesh=mesh,
        scratch_types=[
            pltpu.VMEM((NPT * K,), jnp.int32),
            pltpu.VMEM((NPT * K,), jnp.int32),
            pltpu.VMEM((NBUF, G * K, C), jnp.float32),
            pltpu.VMEM((NBUF, G * K, C), jnp.float32),
            pltpu.VMEM((NBUF, G, C), jnp.float32),
            [pltpu.SemaphoreType.DMA] * NBUF,
            [pltpu.SemaphoreType.DMA] * NBUF,
        ],
    )
    def k(xT_hbm, e_hbm, out_hbm, i0_v, i1_v, r0_v, r1_v, o_v, gsem, osem):
        wid = lax.axis_index("s") * NC + lax.axis_index("c")
        is_tail = wid == NW - 1
        sbase = jnp.where(is_tail, N - NPT, wid * NPT)
        toff = jnp.where(is_tail, NPT - TAIL, 0)
        nch4 = jnp.where(is_tail, TAIL_CH // NBUF, CHUNKS // NBUF)

        pltpu.sync_copy(e_hbm.at[pl.ds(sbase * K, NPT * K)], i0_v)
        pltpu.sync_copy(e_hbm.at[pl.ds(N * K + sbase * K, NPT * K)], i1_v)

        def issue_gather(c, s):
            rows = pl.ds((toff + c * G) * K, G * K)
            pltpu.async_copy(xT_hbm.at[i0_v.at[rows]], r0_v.at[s], gsem[s])
            pltpu.async_copy(xT_hbm.at[i1_v.at[rows]], r1_v.at[s], gsem[s])

        def wait_gather(s):
            dummy = xT_hbm.at[pl.ds(0, G * K)]
            pltpu.make_async_copy(dummy, r0_v.at[s], gsem[s]).wait()
            pltpu.make_async_copy(dummy, r1_v.at[s], gsem[s]).wait()

        def wait_out(s):
            pltpu.make_async_copy(
                o_v.at[s], out_hbm.at[pl.ds(0, G)], osem[s]).wait()

        def compute(c, s, t):
            @pl.when(t > 0)
            def _():
                wait_out(s)

            def node_body(g, carry2):
                base = g * K
                for j in range(C // 16):
                    sl = pl.ds(j * 16, 16)
                    acc = r0_v[s, base, sl] - r1_v[s, base, sl]
                    for kk in range(1, K):
                        acc = jnp.maximum(
                            acc,
                            r0_v[s, base + kk, sl] - r1_v[s, base + kk, sl],
                        )
                    o_v[s, g, sl] = acc
                return carry2

            lax.fori_loop(0, G, node_body, 0, unroll=False)
            pltpu.async_copy(
                o_v.at[s], out_hbm.at[pl.ds(sbase + toff + c * G, G)],
                osem[s])

        nch = nch4 * NBUF
        for s in range(NBUF):
            issue_gather(s, s)

        def body(t, carry):
            for s in range(NBUF):
                c = t * NBUF + s
                wait_gather(s)
                compute(c, s, t)

                @pl.when(c + NBUF < nch)
                def _():
                    issue_gather(c + NBUF, s)
            return carry

        lax.fori_loop(0, nch4, body, 0, unroll=False)

        @pl.when(is_tail)
        def _():
            for s in range(2):
                c = TAIL_CH - 2 + s
                issue_gather(c, s)
                wait_gather(s)
                compute(c, s, 1)

        for s in range(NBUF):
            wait_out(s)

    return k(xT, e2)


def _tr_body(x_ref, o_ref):
    eye = (lax.broadcasted_iota(jnp.int32, (C, C), 0)
           == lax.broadcasted_iota(jnp.int32, (C, C), 1)).astype(jnp.float32)
    o_ref[...] = lax.dot_general(x_ref[...], eye, (((0,), (0,)), ((), ())),
                                 preferred_element_type=jnp.float32)


def _tc_transpose(x2d):
    return pl.pallas_call(
        _tr_body,
        grid=(NBLK,),
        in_specs=[pl.BlockSpec((C, NB), lambda i: (0, i))],
        out_specs=pl.BlockSpec((NB, C), lambda i: (i, 0)),
        out_shape=jax.ShapeDtypeStruct((N, C), jnp.float32),
    )(x2d)


def _mm_body(x_ref, xjT_ref, W1_ref, W2_ref, b_ref, y_ref, s_ref, ss_ref,
             acc_s, acc_ss):
    i = pl.program_id(0)

    @pl.when(i == 0)
    def _():
        acc_s[...] = jnp.zeros_like(acc_s)
        acc_ss[...] = jnp.zeros_like(acc_ss)

    y = lax.dot_general(W1_ref[...], x_ref[...], (((1,), (0,)), ((), ())),
                        preferred_element_type=jnp.float32)
    y += lax.dot_general(W2_ref[...], xjT_ref[...], (((1,), (1,)), ((), ())),
                         preferred_element_type=jnp.float32)
    y += b_ref[...]
    y_ref[...] = y

    col = i * NB + lax.broadcasted_iota(jnp.int32, (1, NB), 1)
    ym = jnp.where(col < N, y, 0.0)
    acc_s[...] += jnp.sum(ym, axis=1, keepdims=True)
    acc_ss[...] += jnp.sum(ym * ym, axis=1, keepdims=True)

    @pl.when(i == NBLK - 1)
    def _():
        s_ref[...] = acc_s[...]
        ss_ref[...] = acc_ss[...]


def _mm_stats(x2d, xjT, W1, W2, b2):
    return pl.pallas_call(
        _mm_body,
        grid=(NBLK,),
        in_specs=[
            pl.BlockSpec((C, NB), lambda i: (0, i)),
            pl.BlockSpec((NB, C), lambda i: (i, 0)),
            pl.BlockSpec((C_OUT, C), lambda i: (0, 0)),
            pl.BlockSpec((C_OUT, C), lambda i: (0, 0)),
            pl.BlockSpec((C_OUT, 1), lambda i: (0, 0)),
        ],
        out_specs=[
            pl.BlockSpec((C_OUT, NB), lambda i: (0, i)),
            pl.BlockSpec((C_OUT, 1), lambda i: (0, 0)),
            pl.BlockSpec((C_OUT, 1), lambda i: (0, 0)),
        ],
        out_shape=[
            jax.ShapeDtypeStruct((C_OUT, N), jnp.float32),
            jax.ShapeDtypeStruct((C_OUT, 1), jnp.float32),
            jax.ShapeDtypeStruct((C_OUT, 1), jnp.float32),
        ],
        scratch_shapes=[
            pltpu.VMEM((C_OUT, 1), jnp.float32),
            pltpu.VMEM((C_OUT, 1), jnp.float32),
        ],
    )(x2d, xjT, W1, W2, b2)


def _bn_body(y_ref, s_ref, ss_ref, g_ref, be_ref, o_ref):
    mean = s_ref[...] * (1.0 / N)
    var = ss_ref[...] * (1.0 / N) - mean * mean
    inv = lax.rsqrt(var + EPS) * g_ref[...]
    shift = be_ref[...] - mean * inv
    o_ref[...] = jnp.maximum(y_ref[...] * inv + shift, 0.0)


def _bn_relu(y, s, ss, g2, be2):
    return pl.pallas_call(
        _bn_body,
        grid=(NBLK,),
        in_specs=[
            pl.BlockSpec((C_OUT, NB), lambda i: (0, i)),
            pl.BlockSpec((C_OUT, 1), lambda i: (0, 0)),
            pl.BlockSpec((C_OUT, 1), lambda i: (0, 0)),
            pl.BlockSpec((C_OUT, 1), lambda i: (0, 0)),
            pl.BlockSpec((C_OUT, 1), lambda i: (0, 0)),
        ],
        out_specs=pl.BlockSpec((C_OUT, NB), lambda i: (0, i)),
        out_shape=jax.ShapeDtypeStruct((C_OUT, N), jnp.float32),
    )(y, s, ss, g2, be2)


def kernel(x, edge_index, W, b, gamma, beta):
    x2d = x[0, :, :, 0]
    xT = _tc_transpose(x2d)

    xjT = _sc_mrconv(xT, edge_index.astype(jnp.int32).reshape(2 * N * K))

    Wd = W.reshape(C_OUT, C, 2)
    y, s, ss = _mm_stats(x2d, xjT, Wd[:, :, 0], Wd[:, :, 1],
                         b.reshape(C_OUT, 1))
    out2d = _bn_relu(y, s, ss, gamma.reshape(C_OUT, 1), beta.reshape(C_OUT, 1))
    return out2d[None, :, :, None]


# speedup vs baseline: 1.0908x; 1.0908x over previous
import functools

import jax
import jax.numpy as jnp
from jax import lax
from jax.experimental import pallas as pl
from jax.experimental.pallas import tpu as pltpu
from jax.experimental.pallas import tpu_sc as plsc

B, C, N, K = 1, 128, 50000, 9
C_OUT = 256

NC, NS = 2, 16
NW = NC * NS
NPT = 1568
TAIL = N - 31 * NPT
G = 8
CHUNKS = NPT // G
TAIL_CH = TAIL // G
NBUF = 4
NB = 2048
NBLK = -(-N // NB)
EPS = 1e-5


def _sc_mrconv(xT, e2):
    mesh = plsc.VectorSubcoreMesh(core_axis_name="c", subcore_axis_name="s")

    @functools.partial(
        pl.kernel,
        out_type=jax.ShapeDtypeStruct((N, C), jnp.float32),
        mesh=mesh,
        scratch_types=[
            pltpu.VMEM((NPT * K,), jnp.int32),
            pltpu.VMEM((NPT * K,), jnp.int32),
            pltpu.VMEM((NBUF, G * K, C), jnp.float32),
            pltpu.VMEM((NBUF, G * K, C), jnp.float32),
            pltpu.VMEM((NBUF, G, C), jnp.float32),
            [pltpu.SemaphoreType.DMA] * NBUF,
            [pltpu.SemaphoreType.DMA] * NBUF,
        ],
    )
    def k(xT_hbm, e_hbm, out_hbm, i0_v, i1_v, r0_v, r1_v, o_v, gsem, osem):
        wid = lax.axis_index("s") * NC + lax.axis_index("c")
        is_tail = wid == NW - 1
        sbase = jnp.where(is_tail, N - NPT, wid * NPT)
        toff = jnp.where(is_tail, NPT - TAIL, 0)
        nch4 = jnp.where(is_tail, TAIL_CH // NBUF, CHUNKS // NBUF)

        pltpu.sync_copy(e_hbm.at[pl.ds(sbase * K, NPT * K)], i0_v)
        pltpu.sync_copy(e_hbm.at[pl.ds(N * K + sbase * K, NPT * K)], i1_v)

        def issue_gather(c, s):
            rows = pl.ds((toff + c * G) * K, G * K)
            pltpu.async_copy(xT_hbm.at[i0_v.at[rows]], r0_v.at[s], gsem[s])
            pltpu.async_copy(xT_hbm.at[i1_v.at[rows]], r1_v.at[s], gsem[s])

        def wait_gather(s):
            dummy = xT_hbm.at[pl.ds(0, G * K)]
            pltpu.make_async_copy(dummy, r0_v.at[s], gsem[s]).wait()
            pltpu.make_async_copy(dummy, r1_v.at[s], gsem[s]).wait()

        def wait_out(s):
            pltpu.make_async_copy(
                o_v.at[s], out_hbm.at[pl.ds(0, G)], osem[s]).wait()

        def compute(c, s, t):
            @pl.when(t > 0)
            def _():
                wait_out(s)

            def node_body(g, carry2):
                base = g * K
                for j in range(C // 16):
                    sl = pl.ds(j * 16, 16)
                    acc = r0_v[s, base, sl] - r1_v[s, base, sl]
                    for kk in range(1, K):
                        acc = jnp.maximum(
                            acc,
                            r0_v[s, base + kk, sl] - r1_v[s, base + kk, sl],
                        )
                    o_v[s, g, sl] = acc
                return carry2

            lax.fori_loop(0, G, node_body, 0, unroll=False)
            pltpu.async_copy(
                o_v.at[s], out_hbm.at[pl.ds(sbase + toff + c * G, G)],
                osem[s])

        nch = nch4 * NBUF
        for s in range(NBUF):
            issue_gather(s, s)

        def body(t, carry):
            for s in range(NBUF):
                c = t * NBUF + s
                wait_gather(s)
                compute(c, s, t)

                @pl.when(c + NBUF < nch)
                def _():
                    issue_gather(c + NBUF, s)
            return carry

        lax.fori_loop(0, nch4, body, 0, unroll=False)

        @pl.when(is_tail)
        def _():
            for s in range(2):
                c = TAIL_CH - 2 + s
                issue_gather(c, s)
                wait_gather(s)
                compute(c, s, 1)

        for s in range(NBUF):
            wait_out(s)

    return k(xT, e2)


def _mm_body(x_ref, xjT_ref, W1_ref, W2_ref, b_ref, y_ref, s_ref, ss_ref,
             acc_s, acc_ss):
    i = pl.program_id(0)

    @pl.when(i == 0)
    def _():
        acc_s[...] = jnp.zeros_like(acc_s)
        acc_ss[...] = jnp.zeros_like(acc_ss)

    y = lax.dot_general(W1_ref[...], x_ref[...], (((1,), (0,)), ((), ())),
                        preferred_element_type=jnp.float32)
    y += lax.dot_general(W2_ref[...], xjT_ref[...], (((1,), (1,)), ((), ())),
                         preferred_element_type=jnp.float32)
    y += b_ref[...]
    y_ref[...] = y.astype(jnp.bfloat16)

    col = i * NB + lax.broadcasted_iota(jnp.int32, (1, NB), 1)
    ym = jnp.where(col < N, y, 0.0)
    acc_s[...] += jnp.sum(ym, axis=1, keepdims=True)
    acc_ss[...] += jnp.sum(ym * ym, axis=1, keepdims=True)

    @pl.when(i == NBLK - 1)
    def _():
        s_ref[...] = acc_s[...]
        ss_ref[...] = acc_ss[...]


def _mm_stats(x2d, xjT, W1, W2, b2):
    return pl.pallas_call(
        _mm_body,
        grid=(NBLK,),
        in_specs=[
            pl.BlockSpec((C, NB), lambda i: (0, i)),
            pl.BlockSpec((NB, C), lambda i: (i, 0)),
            pl.BlockSpec((C_OUT, C), lambda i: (0, 0)),
            pl.BlockSpec((C_OUT, C), lambda i: (0, 0)),
            pl.BlockSpec((C_OUT, 1), lambda i: (0, 0)),
        ],
        out_specs=[
            pl.BlockSpec((C_OUT, NB), lambda i: (0, i)),
            pl.BlockSpec((C_OUT, 1), lambda i: (0, 0)),
            pl.BlockSpec((C_OUT, 1), lambda i: (0, 0)),
        ],
        out_shape=[
            jax.ShapeDtypeStruct((C_OUT, N), jnp.bfloat16),
            jax.ShapeDtypeStruct((C_OUT, 1), jnp.float32),
            jax.ShapeDtypeStruct((C_OUT, 1), jnp.float32),
        ],
        scratch_shapes=[
            pltpu.VMEM((C_OUT, 1), jnp.float32),
            pltpu.VMEM((C_OUT, 1), jnp.float32),
        ],
    )(x2d, xjT, W1, W2, b2)


def _bn_body(y_ref, s_ref, ss_ref, g_ref, be_ref, o_ref):
    mean = s_ref[...] * (1.0 / N)
    var = ss_ref[...] * (1.0 / N) - mean * mean
    inv = lax.rsqrt(var + EPS) * g_ref[...]
    shift = be_ref[...] - mean * inv
    o_ref[...] = jnp.maximum(
        y_ref[...].astype(jnp.float32) * inv + shift, 0.0)


def _bn_relu(y, s, ss, g2, be2):
    return pl.pallas_call(
        _bn_body,
        grid=(NBLK,),
        in_specs=[
            pl.BlockSpec((C_OUT, NB), lambda i: (0, i)),
            pl.BlockSpec((C_OUT, 1), lambda i: (0, 0)),
            pl.BlockSpec((C_OUT, 1), lambda i: (0, 0)),
            pl.BlockSpec((C_OUT, 1), lambda i: (0, 0)),
            pl.BlockSpec((C_OUT, 1), lambda i: (0, 0)),
        ],
        out_specs=pl.BlockSpec((C_OUT, NB), lambda i: (0, i)),
        out_shape=jax.ShapeDtypeStruct((C_OUT, N), jnp.float32),
    )(y, s, ss, g2, be2)


def kernel(x, edge_index, W, b, gamma, beta):
    x2d = x[0, :, :, 0]
    xT = x2d.T

    xjT = _sc_mrconv(xT, edge_index.astype(jnp.int32).reshape(2 * N * K))

    Wd = W.reshape(C_OUT, C, 2)
    y, s, ss = _mm_stats(x2d, xjT, Wd[:, :, 0], Wd[:, :, 1],
                         b.reshape(C_OUT, 1))
    out2d = _bn_relu(y, s, ss, gamma.reshape(C_OUT, 1), beta.reshape(C_OUT, 1))
    return out2d[None, :, :, None]
